# baseline (speedup 1.0000x reference)
import functools

import jax
import jax.numpy as jnp
from jax.experimental import pallas as pl
from jax.experimental.pallas import tpu as pltpu

_B, _N, _C = 8, 20000, 80
_K = 300
_LANES = 128
_ROWS = (_N * _C) // _LANES


def _rowmax_body(x_ref, o_ref):
    o_ref[0, 0, :] = jnp.max(x_ref[0], axis=-1)


def _gather_rows_body(rows_ref, x_ref, o_ref):
    b = pl.program_id(0)
    j = pl.program_id(1)
    sub = rows_ref[b, j] % 8
    o_ref[0, 0, :, :] = x_ref[0, pl.ds(sub, 1), :]


def _gather_candidate_rows(rows, flat):
    out = pl.pallas_call(
        _gather_rows_body,
        grid_spec=pltpu.PrefetchScalarGridSpec(
            num_scalar_prefetch=1,
            grid=(_B, _K),
            in_specs=[
                pl.BlockSpec((1, 8, _LANES), lambda b, j, rows: (b, rows[b, j] // 8, 0))
            ],
            out_specs=pl.BlockSpec((1, 1, 1, _LANES), lambda b, j, rows: (b, j, 0, 0)),
        ),
        out_shape=jax.ShapeDtypeStruct((_B, _K, 1, _LANES), jnp.float32),
    )(rows, flat)
    return out.reshape(_B, _K, _LANES)


@jax.jit
def kernel(pred_logits, pred_boxes, orig_target_sizes):
    flat = pred_logits.reshape(_B, _ROWS, _LANES)
    rowmax = pl.pallas_call(
        _rowmax_body,
        grid=(_B,),
        in_specs=[pl.BlockSpec((1, _ROWS, _LANES), lambda b: (b, 0, 0))],
        out_specs=pl.BlockSpec((1, 1, _ROWS), lambda b: (b, 0, 0)),
        out_shape=jax.ShapeDtypeStruct((_B, 1, _ROWS), jnp.float32),
    )(flat)
    rowmax = rowmax.reshape(_B, _ROWS)

    _, rows = jax.lax.top_k(rowmax, _K)
    rows = jnp.sort(rows, axis=1)
    cand = _gather_candidate_rows(rows, flat)
    vals, pos = jax.lax.top_k(cand.reshape(_B, _K * _LANES), _K)
    row_in_cand = pos // _LANES
    lane = pos % _LANES
    flatidx = jnp.take_along_axis(rows, row_in_cand, axis=1) * _LANES + lane

    labels = flatidx % _C
    qidx = flatidx // _C
    scores = jax.nn.sigmoid(vals)

    bx = jnp.take_along_axis(pred_boxes, qidx[:, :, None], axis=1)
    cx, cy, w, h = jnp.split(bx, 4, axis=-1)
    xyxy = jnp.concatenate(
        [cx - 0.5 * w, cy - 0.5 * h, cx + 0.5 * w, cy + 0.5 * h], axis=-1
    )
    scale = jnp.tile(orig_target_sizes.astype(jnp.float32), (1, 2))[:, None, :]
    boxes = xyxy * scale
    return labels, boxes, scores


# speedup vs baseline: 3.1544x; 3.1544x over previous
import functools

import jax
import jax.numpy as jnp
from jax.experimental import pallas as pl

_B, _N, _C = 8, 20000, 80
_K = 300
_LANES = 128
_ROWS = (_N * _C) // _LANES


def _rowmax_body(x_ref, o_ref):
    o_ref[0, 0, :] = jnp.max(x_ref[0], axis=-1)


@jax.jit
def kernel(pred_logits, pred_boxes, orig_target_sizes):
    qmax = pl.pallas_call(
        _rowmax_body,
        grid=(_B,),
        in_specs=[pl.BlockSpec((1, _N, _C), lambda b: (b, 0, 0))],
        out_specs=pl.BlockSpec((1, 1, _N), lambda b: (b, 0, 0)),
        out_shape=jax.ShapeDtypeStruct((_B, 1, _N), jnp.float32),
    )(pred_logits)
    qmax = qmax.reshape(_B, _N)

    _, qrows = jax.lax.top_k(qmax, _K)
    qrows = jnp.sort(qrows, axis=1)
    cand = jnp.take_along_axis(pred_logits, qrows[:, :, None], axis=1)
    vals, pos = jax.lax.top_k(cand.reshape(_B, _K * _C), _K)
    row_in_cand = pos // _C
    labels = pos % _C
    qidx = jnp.take_along_axis(qrows, row_in_cand, axis=1)
    scores = jax.nn.sigmoid(vals)

    bx = jnp.take_along_axis(pred_boxes, qidx[:, :, None], axis=1)
    cx, cy, w, h = jnp.split(bx, 4, axis=-1)
    xyxy = jnp.concatenate(
        [cx - 0.5 * w, cy - 0.5 * h, cx + 0.5 * w, cy + 0.5 * h], axis=-1
    )
    scale = jnp.tile(orig_target_sizes.astype(jnp.float32), (1, 2))[:, None, :]
    boxes = xyxy * scale
    return labels, boxes, scores
